# baseline (speedup 1.0000x reference)
import jax
import jax.numpy as jnp
from jax.experimental import pallas as pl
from jax.experimental.pallas import tpu as pltpu

C_IN, H_IN, W_IN = 3, 32, 32
K = 5
C1, H1, W1 = 6, 28, 28
C2, H2, W2 = 16, 10, 10
PH1 = 14
PH2, PW2 = 5, 5
FC1, FC2, NCLS = 120, 84, 10

L1 = W1 * C1
L1P = L1 - C1
L2 = W2 * C2
L2P = L2 - C2
KC1 = C_IN * K * W_IN
KC2 = K * L1P
KF1 = PH2 * L2P


def _fused_kernel(x_ref, t1_ref, b1_ref, t2_ref, b2_ref, f1_ref, bf1_ref,
                  w2_ref, bf2_ref, w3_ref, b3_ref, o_ref, sx_ref, sp1_ref):
    f32 = jnp.float32
    bf16 = jnp.bfloat16
    B = x_ref.shape[1]

    sx_ref[...] = x_ref[...].reshape(B * H_IN, C_IN * W_IN)
    lhs1 = jnp.concatenate(
        [sx_ref[pl.ds(kh * B, H1 * B), :] for kh in range(K)], axis=-1)
    c1 = jnp.dot(lhs1, t1_ref[...],
                 preferred_element_type=f32) + b1_ref[...]
    c1 = jnp.maximum(c1, 0.0).astype(bf16)

    wm1 = jnp.maximum(c1[:, 0:L1P], c1[:, C1:L1])
    wm1 = wm1.reshape(PH1, 2 * B, L1P)
    p1 = jnp.maximum(wm1[:, 0:B, :], wm1[:, B:2 * B, :]
                     ).reshape(PH1 * B, L1P)

    sp1_ref[...] = p1
    lhs2 = jnp.concatenate(
        [sp1_ref[pl.ds(kh * B, H2 * B), :] for kh in range(K)], axis=-1)
    c2 = jnp.dot(lhs2, t2_ref[...],
                 preferred_element_type=f32) + b2_ref[...]
    c2 = jnp.maximum(c2, 0.0).astype(bf16)

    wm2 = jnp.maximum(c2[:, 0:L2P], c2[:, C2:L2])
    wm2 = wm2.reshape(PH2, 2 * B, L2P)
    p2 = jnp.maximum(wm2[:, 0:B, :], wm2[:, B:2 * B, :])

    lhs3 = jnp.concatenate([p2[r] for r in range(PH2)], axis=-1)
    y1 = jnp.maximum(jnp.dot(lhs3, f1_ref[...], preferred_element_type=f32)
                     + bf1_ref[...], 0.0).astype(bf16)
    y2 = jnp.maximum(jnp.dot(y1, w2_ref[...], preferred_element_type=f32)
                     + bf2_ref[...], 0.0).astype(bf16)
    o_ref[...] = (jnp.dot(y2, w3_ref[...], preferred_element_type=f32)
                  + b3_ref[...])


def kernel(x, t1, b1, t2, b2, f1, bf1, w2, bf2, w3, b3):
    n = x.shape[0]
    B = next(b for b in (256, 128, 64, 32, 16, 8, 4, 2, 1) if n % b == 0)

    bf16 = jnp.bfloat16
    xt = x.transpose(2, 0, 1, 3).reshape(H_IN, n, C_IN * W_IN).astype(bf16)
    T1 = t1.reshape(KC1, L1).astype(bf16)
    T2 = t2.reshape(KC2, L2).astype(bf16)
    F1 = f1.reshape(KF1, FC1).astype(bf16)
    W2 = w2.astype(bf16)
    W3 = w3.astype(bf16)

    def full(shape):
        return pl.BlockSpec(shape, lambda i: (0,) * len(shape))

    out = pl.pallas_call(
        _fused_kernel,
        out_shape=jax.ShapeDtypeStruct((n, NCLS), jnp.float32),
        grid=(n // B,),
        in_specs=[
            pl.BlockSpec((H_IN, B, C_IN * W_IN), lambda i: (0, i, 0)),
            full((KC1, L1)),
            full((1, L1)),
            full((KC2, L2)),
            full((1, L2)),
            full((KF1, FC1)),
            full((1, FC1)),
            full((FC1, FC2)),
            full((1, FC2)),
            full((FC2, NCLS)),
            full((1, NCLS)),
        ],
        out_specs=pl.BlockSpec((B, NCLS), lambda i: (i, 0)),
        scratch_shapes=[
            pltpu.VMEM((H_IN * B, C_IN * W_IN), bf16),
            pltpu.VMEM((PH1 * B, L1P), bf16),
        ],
        compiler_params=pltpu.CompilerParams(
            dimension_semantics=("arbitrary",)),
    )(xt, T1, b1, T2, b2, F1, bf1, W2, bf2, W3, b3)
    return out


# speedup vs baseline: 1.0742x; 1.0742x over previous
import jax
import jax.numpy as jnp
from jax.experimental import pallas as pl
from jax.experimental.pallas import tpu as pltpu

C_IN, H_IN, W_IN = 3, 32, 32
K = 5
C1, H1, W1 = 6, 28, 28
C2, H2, W2 = 16, 10, 10
PH1 = 14
PH2, PW2 = 5, 5
FC1, FC2, NCLS = 120, 84, 10

L1 = W1 * C1
L1P = L1 - C1
L2 = W2 * C2
L2P = L2 - C2
KC1 = C_IN * K * W_IN
KC2 = K * L1P
KF1 = PH2 * L2P


def _fused_kernel(x_ref, t1_ref, b1_ref, t2_ref, b2_ref, f1_ref, bf1_ref,
                  w2_ref, bf2_ref, w3_ref, b3_ref, o_ref):
    f32 = jnp.float32
    bf16 = jnp.bfloat16
    B = x_ref.shape[1]

    xv = x_ref[...]
    lhs1 = jnp.concatenate(
        [xv[kh:kh + H1].reshape(H1 * B, C_IN * W_IN) for kh in range(K)],
        axis=-1)
    c1 = jnp.dot(lhs1, t1_ref[...],
                 preferred_element_type=f32) + b1_ref[...]
    c1 = jnp.maximum(c1, 0.0).astype(bf16)

    wm1 = jnp.maximum(c1[:, 0:L1P], c1[:, C1:L1])
    wm1 = wm1.reshape(PH1, 2 * B, L1P)
    p1 = jnp.maximum(wm1[:, 0:B, :], wm1[:, B:2 * B, :]
                     ).reshape(PH1 * B, L1P)

    p13 = p1.reshape(PH1, B, L1P)
    lhs2 = jnp.concatenate(
        [p13[kh:kh + H2].reshape(H2 * B, L1P) for kh in range(K)], axis=-1)
    c2 = jnp.dot(lhs2, t2_ref[...],
                 preferred_element_type=f32) + b2_ref[...]
    c2 = jnp.maximum(c2, 0.0).astype(bf16)

    wm2 = jnp.maximum(c2[:, 0:L2P], c2[:, C2:L2])
    wm2 = wm2.reshape(PH2, 2 * B, L2P)
    p2 = jnp.maximum(wm2[:, 0:B, :], wm2[:, B:2 * B, :])

    lhs3 = jnp.concatenate([p2[r] for r in range(PH2)], axis=-1)
    y1 = jnp.maximum(jnp.dot(lhs3, f1_ref[...], preferred_element_type=f32)
                     + bf1_ref[...], 0.0).astype(bf16)
    y2 = jnp.maximum(jnp.dot(y1, w2_ref[...], preferred_element_type=f32)
                     + bf2_ref[...], 0.0).astype(bf16)
    o_ref[...] = (jnp.dot(y2, w3_ref[...], preferred_element_type=f32)
                  + b3_ref[...])


def kernel(x, t1, b1, t2, b2, f1, bf1, w2, bf2, w3, b3):
    n = x.shape[0]
    B = next(b for b in (512, 256, 128, 64, 32, 16, 8, 4, 2, 1) if n % b == 0)

    bf16 = jnp.bfloat16
    xt = x.transpose(2, 0, 1, 3).reshape(H_IN, n, C_IN * W_IN).astype(bf16)
    T1 = t1.reshape(KC1, L1).astype(bf16)
    T2 = t2.reshape(KC2, L2).astype(bf16)
    F1 = f1.reshape(KF1, FC1).astype(bf16)
    W2 = w2.astype(bf16)
    W3 = w3.astype(bf16)

    def full(shape):
        return pl.BlockSpec(shape, lambda i: (0,) * len(shape))

    out = pl.pallas_call(
        _fused_kernel,
        out_shape=jax.ShapeDtypeStruct((n, NCLS), jnp.float32),
        grid=(n // B,),
        in_specs=[
            pl.BlockSpec((H_IN, B, C_IN * W_IN), lambda i: (0, i, 0)),
            full((KC1, L1)),
            full((1, L1)),
            full((KC2, L2)),
            full((1, L2)),
            full((KF1, FC1)),
            full((1, FC1)),
            full((FC1, FC2)),
            full((1, FC2)),
            full((FC2, NCLS)),
            full((1, NCLS)),
        ],
        out_specs=pl.BlockSpec((B, NCLS), lambda i: (i, 0)),
        compiler_params=pltpu.CompilerParams(
            dimension_semantics=("arbitrary",)),
    )(xt, T1, b1, T2, b2, F1, bf1, W2, bf2, W3, b3)
    return out
